# baseline (speedup 1.0000x reference)
import functools

import jax
import jax.numpy as jnp
from jax import lax
from jax.experimental import pallas as pl
from jax.experimental.pallas import tpu as pltpu
from jax.experimental.pallas import tpu_sc as plsc

D_MODEL_C = 1024
D_LAT_C = 4096
K_C = 32
ROWS = 2048
BLK = 256

_NW = 32
_RPW = ROWS // _NW


def _enc_body(x_ref, we_ref, b1_ref, b2_ref, pre_ref):
    x = x_ref[...]
    pre = jax.lax.dot_general(
        x, we_ref[...], (((1,), (1,)), ((), ())),
        preferred_element_type=jnp.float32)
    pre_ref[...] = pre + b1_ref[...] + b2_ref[...]


def _sort16(v):
    return jnp.sort(v)


def _merge16(a, b):
    rb = lax.rev(b, (0,))
    lo = jnp.minimum(a, rb)
    hi = jnp.maximum(a, rb)
    return _sort16(lo), _sort16(hi)


def _top32_sorted(a, b):
    a_lo, a_hi = a
    b_lo, b_hi = b
    m0 = jnp.maximum(a_lo, lax.rev(b_hi, (0,)))
    m1 = jnp.maximum(a_hi, lax.rev(b_lo, (0,)))
    lo = jnp.minimum(m0, m1)
    hi = jnp.maximum(m0, m1)
    return _sort16(lo), _sort16(hi)


def _min_top32_of4(v0, v1, v2, v3):
    a = _merge16(_sort16(v0), _sort16(v1))
    b = _merge16(_sort16(v2), _sort16(v3))
    m0 = jnp.maximum(a[0], lax.rev(b[1], (0,)))
    m1 = jnp.maximum(a[1], lax.rev(b[0], (0,)))
    return jnp.min(jnp.minimum(m0, m1))


def _f2key(v):
    bits = plsc.bitcast(v, jnp.uint32)
    neg = bits >= jnp.uint32(0x80000000)
    return jnp.where(neg, ~bits, bits | jnp.uint32(0x80000000))


def _sc_body(pre_hbm, thr_hbm, row_v, cand_v, thr_v):
    wid = lax.axis_index("s") * 2 + lax.axis_index("c")
    base = wid * _RPW

    def row_body(i, _carry):
        pltpu.sync_copy(pre_hbm.at[base + i], row_v)

        def ga(k, acc):
            a0, a1, a2, a3 = acc
            o = k * 64
            return (jnp.maximum(a0, row_v[pl.ds(o, 16)]),
                    jnp.maximum(a1, row_v[pl.ds(o + 16, 16)]),
                    jnp.maximum(a2, row_v[pl.ds(o + 32, 16)]),
                    jnp.maximum(a3, row_v[pl.ds(o + 48, 16)]))

        init = (row_v[pl.ds(0, 16)], row_v[pl.ds(16, 16)],
                row_v[pl.ds(32, 16)], row_v[pl.ds(48, 16)])
        a0, a1, a2, a3 = lax.fori_loop(1, 64, ga, init, unroll=8)

        tau1 = _min_top32_of4(a0, a1, a2, a3)
        tau1v = jnp.full((16,), tau1, jnp.float32)

        neginf = jnp.full((16,), -jnp.inf, jnp.float32)
        for j in range(9):
            cand_v[pl.ds(j * 16, 16)] = neginf

        def fb(k, off_v):
            v = row_v[pl.ds(k * 16, 16)]
            m = v >= tau1v
            pos = off_v + plsc.cumsum(m.astype(jnp.int32)) - 1
            pos = jnp.where(m, pos, 0)
            plsc.store_scatter(cand_v, [pos], v, mask=m)
            return off_v + plsc.all_reduce_population_count(m)

        off_v = lax.fori_loop(0, 256, fb, jnp.zeros((16,), jnp.int32),
                              unroll=8)
        off = jnp.max(off_v)

        def small_case(off_s):
            del off_s
            s = [_sort16(cand_v[pl.ds(j * 16, 16)]) for j in range(8)]
            p1 = _merge16(s[0], s[1])
            p2 = _merge16(s[2], s[3])
            p3 = _merge16(s[4], s[5])
            p4 = _merge16(s[6], s[7])
            t12 = _top32_sorted(p1, p2)
            t34 = _top32_sorted(p3, p4)
            m0 = jnp.maximum(t12[0], lax.rev(t34[1], (0,)))
            m1 = jnp.maximum(t12[1], lax.rev(t34[0], (0,)))
            return jnp.min(jnp.minimum(m0, m1))

        def big_case(off_s):
            nv = (off_s + 15) // 16
            lanes = lax.iota(jnp.int32, 16)

            def bit_step(b, pref):
                bitv = lax.shift_left(
                    jnp.full((16,), 1, jnp.uint32),
                    jnp.full((16,), 31 - b, jnp.int32).astype(jnp.uint32))
                cand = pref | bitv

                def inner(j, cnt):
                    v = cand_v[pl.ds(j * 16, 16)]
                    kv = _f2key(v)
                    valid = (lanes + j * 16) < off_s
                    m = jnp.logical_and(valid, kv >= cand)
                    return cnt + plsc.all_reduce_population_count(m)

                cnt = lax.fori_loop(0, nv, inner,
                                    jnp.zeros((16,), jnp.int32))
                return jnp.where(cnt >= K_C, cand, pref)

            pref = lax.fori_loop(0, 32, bit_step,
                                 jnp.zeros((16,), jnp.uint32))
            negm = pref < jnp.uint32(0x80000000)
            bits = jnp.where(negm, ~pref, pref ^ jnp.uint32(0x80000000))
            return jnp.min(plsc.bitcast(bits, jnp.float32))

        tau = lax.cond(off <= 128, small_case, big_case, off)

        seg = (i // 16) * 16
        lane = i % 16
        lm = lax.iota(jnp.int32, 16) == lane
        cur = thr_v[pl.ds(seg, 16)]
        thr_v[pl.ds(seg, 16)] = jnp.where(lm, jnp.full((16,), tau), cur)
        return 0

    lax.fori_loop(0, _RPW, row_body, 0)
    pltpu.sync_copy(thr_v, thr_hbm.at[pl.ds(base, _RPW)])


def _dec_body(pre_ref, thr_ref, wd_ref, lat_ref, rec_ref):
    pre = pre_ref[...]
    thr = thr_ref[...]
    lat = jnp.where(pre >= thr, jnp.maximum(pre, 0.0), 0.0)
    lat_ref[...] = lat
    rec_ref[...] = jax.lax.dot_general(
        lat.astype(jnp.bfloat16), wd_ref[...].astype(jnp.bfloat16),
        (((1,), (1,)), ((), ())),
        preferred_element_type=jnp.float32)


def kernel(x, W_enc, b_enc, enc_bias, W_dec):
    B, S, D = x.shape
    x2 = x.reshape(B * S, D)
    b1 = b_enc.reshape(1, D_LAT_C)
    b2 = enc_bias.reshape(1, D_LAT_C)
    grid = (B * S) // BLK

    pre = pl.pallas_call(
        _enc_body,
        grid=(grid,),
        in_specs=[
            pl.BlockSpec((BLK, D), lambda i: (i, 0)),
            pl.BlockSpec((D_LAT_C, D), lambda i: (0, 0)),
            pl.BlockSpec((1, D_LAT_C), lambda i: (0, 0)),
            pl.BlockSpec((1, D_LAT_C), lambda i: (0, 0)),
        ],
        out_specs=pl.BlockSpec((BLK, D_LAT_C), lambda i: (i, 0)),
        out_shape=jax.ShapeDtypeStruct((B * S, D_LAT_C), jnp.float32),
        compiler_params=pltpu.CompilerParams(
            dimension_semantics=("arbitrary",),
        ),
    )(x2, W_enc, b1, b2)

    mesh = plsc.VectorSubcoreMesh(core_axis_name="c", subcore_axis_name="s")
    thr = pl.kernel(
        _sc_body,
        out_type=jax.ShapeDtypeStruct((ROWS,), jnp.float32),
        mesh=mesh,
        scratch_types=[
            pltpu.VMEM((D_LAT_C,), jnp.float32),
            pltpu.VMEM((D_LAT_C + 128,), jnp.float32),
            pltpu.VMEM((_RPW,), jnp.float32),
        ],
        compiler_params=pltpu.CompilerParams(needs_layout_passes=False),
    )(pre)

    thr2 = thr.reshape(ROWS, 1)

    lat2, rec2 = pl.pallas_call(
        _dec_body,
        grid=(grid,),
        in_specs=[
            pl.BlockSpec((BLK, D_LAT_C), lambda i: (i, 0)),
            pl.BlockSpec((BLK, 1), lambda i: (i, 0)),
            pl.BlockSpec((D, D_LAT_C), lambda i: (0, 0)),
        ],
        out_specs=[
            pl.BlockSpec((BLK, D_LAT_C), lambda i: (i, 0)),
            pl.BlockSpec((BLK, D), lambda i: (i, 0)),
        ],
        out_shape=[
            jax.ShapeDtypeStruct((B * S, D_LAT_C), jnp.float32),
            jax.ShapeDtypeStruct((B * S, D), jnp.float32),
        ],
        compiler_params=pltpu.CompilerParams(
            dimension_semantics=("arbitrary",),
        ),
    )(pre, thr2, W_dec)

    return rec2.reshape(B, S, D), lat2.reshape(B, S, D_LAT_C)


# speedup vs baseline: 2.7125x; 2.7125x over previous
import jax
import jax.numpy as jnp
from jax.experimental import pallas as pl
from jax.experimental.pallas import tpu as pltpu

D_MODEL_C = 1024
D_LAT_C = 4096
K_C = 32
ROWS = 2048
BLK = 256


def _body(x_ref, we_ref, b1_ref, b2_ref, wd_ref, lat_ref, rec_ref):
    x = x_ref[...]
    pre = jax.lax.dot_general(
        x, we_ref[...], (((1,), (1,)), ((), ())),
        preferred_element_type=jnp.float32)
    pre = pre + b1_ref[...] + b2_ref[...]

    def f2key(v):
        bits = jax.lax.bitcast_convert_type(v, jnp.uint32)
        neg = bits >= jnp.uint32(0x80000000)
        return jnp.where(neg, ~bits, bits | jnp.uint32(0x80000000))

    key = f2key(pre)

    ch = jnp.max(pre.reshape(BLK, 32, 128), axis=2)
    tau0 = jnp.min(ch, axis=1, keepdims=True)
    rmax = jnp.max(ch, axis=1, keepdims=True)
    k0 = f2key(tau0)
    k0 = jnp.where(k0 >= jnp.uint32(1), k0 - jnp.uint32(1), jnp.uint32(0))
    kM = f2key(rmax)
    rng = kM - k0
    e = (jax.lax.bitcast_convert_type(rng.astype(jnp.float32), jnp.int32)
         >> 23) - 127
    shift = jnp.clip(31 - e, 0, 31).astype(jnp.uint32)
    keyn = jnp.where(key > k0,
                     jax.lax.shift_left(key - k0, shift),
                     jnp.uint32(0))

    kh = (jax.lax.shift_right_logical(keyn, jnp.uint32(16))
          ^ jnp.uint32(0x8000)).astype(jnp.uint16).astype(jnp.int16)

    def step_hi(b, carry):
        prefix, cnt_at = carry
        bit = jax.lax.shift_left(jnp.uint32(1), jnp.uint32(31) - b.astype(jnp.uint32))
        cand = prefix | bit
        cand16 = (jax.lax.shift_right_logical(cand, jnp.uint32(16))
                  ^ jnp.uint32(0x8000)).astype(jnp.uint16).astype(jnp.int16)
        t = (kh >= cand16).astype(jnp.int16)
        for w in (2048, 1024, 512, 256, 128):
            t = t[:, :w] + t[:, w:]
        cnt = jnp.sum(t.astype(jnp.int32), axis=1, keepdims=True)
        take = cnt >= K_C
        return (jnp.where(take, cand, prefix), jnp.where(take, cnt, cnt_at))

    def step(b, carry):
        prefix, cnt_at = carry
        bit = jax.lax.shift_left(jnp.uint32(1), jnp.uint32(31) - b.astype(jnp.uint32))
        cand = prefix | bit
        cnt = jnp.sum((keyn >= cand).astype(jnp.int32), axis=1, keepdims=True)
        take = cnt >= K_C
        return (jnp.where(take, cand, prefix), jnp.where(take, cnt, cnt_at))

    carry0 = (jnp.zeros((BLK, 1), jnp.uint32),
              jnp.full((BLK, 1), D_LAT_C, jnp.int32))
    carry = jax.lax.fori_loop(0, 16, step_hi, carry0, unroll=True)

    def finish(c):
        return jax.lax.fori_loop(16, 32, step, c, unroll=True)

    prefix, _ = jax.lax.cond(
        jnp.any(carry[1] != K_C), finish, lambda c: c, carry)

    lat = jnp.where(keyn >= prefix, jnp.maximum(pre, 0.0), 0.0)
    lat_ref[...] = lat
    rec_ref[...] = jax.lax.dot_general(
        lat.astype(jnp.bfloat16), wd_ref[...].astype(jnp.bfloat16),
        (((1,), (1,)), ((), ())),
        preferred_element_type=jnp.float32)


def kernel(x, W_enc, b_enc, enc_bias, W_dec):
    B, S, D = x.shape
    x2 = x.reshape(B * S, D)
    b1 = b_enc.reshape(1, D_LAT_C)
    b2 = enc_bias.reshape(1, D_LAT_C)
    grid = (B * S) // BLK

    lat2, rec2 = pl.pallas_call(
        _body,
        grid=(grid,),
        in_specs=[
            pl.BlockSpec((BLK, D), lambda i: (i, 0)),
            pl.BlockSpec((D_LAT_C, D), lambda i: (0, 0)),
            pl.BlockSpec((1, D_LAT_C), lambda i: (0, 0)),
            pl.BlockSpec((1, D_LAT_C), lambda i: (0, 0)),
            pl.BlockSpec((D, D_LAT_C), lambda i: (0, 0)),
        ],
        out_specs=[
            pl.BlockSpec((BLK, D_LAT_C), lambda i: (i, 0)),
            pl.BlockSpec((BLK, D), lambda i: (i, 0)),
        ],
        out_shape=[
            jax.ShapeDtypeStruct((B * S, D_LAT_C), jnp.float32),
            jax.ShapeDtypeStruct((B * S, D), jnp.float32),
        ],
        compiler_params=pltpu.CompilerParams(
            dimension_semantics=("arbitrary",),
        ),
    )(x2, W_enc, b1, b2, W_dec)

    return rec2.reshape(B, S, D), lat2.reshape(B, S, D_LAT_C)
